# baseline (speedup 1.0000x reference)
import functools

import jax
import jax.numpy as jnp
from jax import lax
from jax.experimental import pallas as pl
from jax.experimental.pallas import tpu as pltpu
from jax.experimental.pallas import tpu_sc as plsc

N = 10000
D = 256
H = D // 2
NC = 2
NS = 16
L = 16
C = 80
NI = 6
NRB = 4
NU = 12
BR = 640
BR_LAST = N - (NS - 1) * BR


def _sc_segment_sum(ego2, src, dst, w, zrows, nchunks):
    T = nchunks * C

    mesh = plsc.VectorSubcoreMesh(
        core_axis_name="c", subcore_axis_name="s", num_cores=NC,
        num_subcores=NS)

    @functools.partial(
        pl.kernel,
        out_type=jax.ShapeDtypeStruct((2 * N, H), jnp.float32),
        mesh=mesh,
        scratch_types=(
            [pltpu.VMEM((C, H), jnp.float32) for _ in range(NRB)]
            + [pltpu.VMEM((C,), jnp.int32) for _ in range(NI)]
            + [pltpu.VMEM((C,), jnp.int32) for _ in range(NI)]
            + [pltpu.VMEM((C,), jnp.float32) for _ in range(NI)]
            + [pltpu.VMEM_SHARED((N, H), jnp.float32)]
            + [pltpu.SemaphoreType.DMA] * (NI + 2 * NRB)
        ),
    )
    def body(ego2_r, src_r, dst_r, w_r, z_r, lap2_r, *sc):
        rows = sc[0:NRB]
        sidx = sc[NRB:NRB + NI]
        didx = sc[NRB + NI:NRB + 2 * NI]
        wvb = sc[NRB + 2 * NI:NRB + 3 * NI]
        acc = sc[NRB + 3 * NI]
        isem = sc[NRB + 3 * NI + 1:NRB + 4 * NI + 1]
        gsem = sc[NRB + 4 * NI + 1:NRB + 4 * NI + 1 + NRB]
        ssem = sc[NRB + 4 * NI + 1 + NRB:NRB + 4 * NI + 1 + 2 * NRB]

        c = lax.axis_index("c")
        s = lax.axis_index("s")

        @pl.when(s < NS - 1)
        def _():
            pltpu.sync_copy(z_r, acc.at[pl.ds(s * BR, BR)])

        @pl.when(s == NS - 1)
        def _():
            pltpu.sync_copy(z_r.at[pl.ds(0, BR_LAST)],
                            acc.at[pl.ds((NS - 1) * BR, BR_LAST)])

        base_e = s * T
        coff = c * N

        def idx_start(g, m):
            off = base_e + g * C
            pltpu.async_copy(src_r.at[pl.ds(off, C)], sidx[m], isem[m])
            pltpu.async_copy(dst_r.at[pl.ds(off, C)], didx[m], isem[m])
            pltpu.async_copy(w_r.at[pl.ds(off, C)], wvb[m], isem[m])

        def idx_wait(g, m):
            off = base_e + g * C
            pltpu.make_async_copy(
                src_r.at[pl.ds(off, C)], sidx[m], isem[m]).wait()
            pltpu.make_async_copy(
                dst_r.at[pl.ds(off, C)], didx[m], isem[m]).wait()
            pltpu.make_async_copy(
                w_r.at[pl.ds(off, C)], wvb[m], isem[m]).wait()

        def add_off(m):
            for k in range(C // L):
                sidx[m][pl.ds(k * L, L)] = sidx[m][pl.ds(k * L, L)] + coff

        def gather_start(b, m):
            pltpu.async_copy(ego2_r.at[sidx[m]], rows[b], gsem[b])

        def gather_wait(b, m):
            pltpu.make_async_copy(
                ego2_r.at[sidx[m]], rows[b], gsem[b]).wait()

        def scat_start(b, m):
            pltpu.async_copy(rows[b], acc.at[didx[m]], ssem[b], add=True)

        def scat_wait(b, m):
            pltpu.make_async_copy(
                rows[b], acc.at[didx[m]], ssem[b]).wait()

        def scale(b, m):
            def group_body(gg, carry):
                w16 = wvb[m][pl.ds(gg * L, L)]
                for ii in range(L):
                    e = gg * L + ii
                    wvec = jnp.full((L,), w16[ii], dtype=jnp.float32)
                    for j in range(H // L):
                        rows[b][e, pl.ds(j * L, L)] = (
                            rows[b][e, pl.ds(j * L, L)] * wvec)
                return carry

            lax.fori_loop(0, C // L, group_body, 0)

        plsc.subcore_barrier()

        idx_start(0, 0)
        idx_start(1, 1)
        idx_start(2, 2)
        idx_start(3, 3)
        idx_wait(0, 0)
        add_off(0)
        gather_start(0, 0)
        idx_wait(1, 1)
        add_off(1)
        gather_start(1, 1)

        ngrp = nchunks // NU

        def pipe_body(t, carry):
            for i in range(NU):
                g = NU * t + i
                b = i % NRB
                bp2 = (i + NRB - 2) % NRB
                bn = (i + 2) % NRB
                m = i % NI
                m_prev2 = (i + NI - 2) % NI
                m_g2 = (i + 2) % NI
                m_pref = (i + 4) % NI

                if i <= 1:
                    @pl.when(t >= 1)
                    def _():
                        scat_wait(bp2, m_prev2)
                else:
                    scat_wait(bp2, m_prev2)

                def _launch():
                    idx_wait(g + 2, m_g2)
                    add_off(m_g2)
                    gather_start(bn, m_g2)

                if i >= NU - 2:
                    @pl.when(t < ngrp - 1)
                    def _():
                        _launch()
                else:
                    _launch()

                gather_wait(b, m)

                if i <= NU - 5:
                    idx_start(g + 4, m_pref)
                else:
                    @pl.when(t < ngrp - 1)
                    def _():
                        idx_start(g + 4, m_pref)

                scale(b, m)
                scat_start(b, m)
            return carry

        lax.fori_loop(0, ngrp, pipe_body, 0)

        scat_wait((nchunks - 2) % NRB, (nchunks - 2) % NI)
        scat_wait((nchunks - 1) % NRB, (nchunks - 1) % NI)

        plsc.subcore_barrier()

        @pl.when(s < NS - 1)
        def _():
            pltpu.sync_copy(acc.at[pl.ds(s * BR, BR)],
                            lap2_r.at[pl.ds(coff + s * BR, BR)])

        @pl.when(s == NS - 1)
        def _():
            pltpu.sync_copy(acc.at[pl.ds((NS - 1) * BR, BR_LAST)],
                            lap2_r.at[pl.ds(coff + (NS - 1) * BR, BR_LAST)])

    return body(ego2, src, dst, w, zrows)


def _tc_dense(ego, lap2, W1, W2, b):
    R = 1000
    nb = N // R

    def body(ego_r, lapl_r, lapr_r, w1_r, w2_r, b_r, out_r):
        el = ego_r[:, :H]
        er = ego_r[:, H:]
        ll = lapl_r[...]
        lr = lapr_r[...]
        acc = jnp.dot(el + ll, w1_r[:H, :], preferred_element_type=jnp.float32)
        acc += jnp.dot(er + lr, w1_r[H:, :], preferred_element_type=jnp.float32)
        acc += jnp.dot(el * ll, w2_r[:H, :], preferred_element_type=jnp.float32)
        acc += jnp.dot(er * lr, w2_r[H:, :], preferred_element_type=jnp.float32)
        pre = acc + b_r[...]
        y = jnp.where(pre >= 0, pre, 0.2 * pre)
        norm = jnp.sqrt(jnp.sum(y * y, axis=1, keepdims=True))
        out_r[...] = y / jnp.maximum(norm, 1e-12)

    return pl.pallas_call(
        body,
        grid=(nb,),
        in_specs=[
            pl.BlockSpec((R, D), lambda i: (i, 0)),
            pl.BlockSpec((R, H), lambda i: (i, 0)),
            pl.BlockSpec((R, H), lambda i: (i + nb, 0)),
            pl.BlockSpec((D, D), lambda i: (0, 0)),
            pl.BlockSpec((D, D), lambda i: (0, 0)),
            pl.BlockSpec((1, D), lambda i: (0, 0)),
        ],
        out_specs=pl.BlockSpec((R, D), lambda i: (i, 0)),
        out_shape=jax.ShapeDtypeStruct((N, D), jnp.float32),
    )(ego, lap2, lap2, W1, W2, b)


def kernel(ego_embeddings, edge_index, edge_weight, W1, W2, b1, b2):
    E = edge_weight.shape[0]

    ego2 = jnp.concatenate(
        [ego_embeddings[:, :H], ego_embeddings[:, H:]], axis=0)

    per_tile = -(-E // (NS * C * NU)) * C * NU
    Ep = per_tile * NS
    pad = Ep - E
    nchunks = per_tile // C
    src = jnp.concatenate([edge_index[0], jnp.zeros((pad,), jnp.int32)])
    dst = jnp.concatenate([edge_index[1], jnp.zeros((pad,), jnp.int32)])
    w = jnp.concatenate([edge_weight, jnp.zeros((pad,), jnp.float32)])

    zrows = jnp.zeros((BR, H), jnp.float32)
    lap2 = _sc_segment_sum(ego2, src, dst, w, zrows, nchunks)

    b = b1 + b2
    return _tc_dense(ego_embeddings, lap2, W1, W2, b)


# speedup vs baseline: 1.7846x; 1.7846x over previous
import functools

import jax
import jax.numpy as jnp
from jax import lax
from jax.experimental import pallas as pl
from jax.experimental.pallas import tpu as pltpu
from jax.experimental.pallas import tpu_sc as plsc

N = 10000
D = 256
H = D // 2
NC = 2
NS = 16
L = 16
C = 128
NI = 4
NRB = 2
BR = 640
BR_LAST = N - (NS - 1) * BR


def _sc_segment_sum(ego2, src, dst, w, zrows, nchunks):
    T = nchunks * C

    mesh = plsc.VectorSubcoreMesh(
        core_axis_name="c", subcore_axis_name="s", num_cores=NC,
        num_subcores=NS)

    @functools.partial(
        pl.kernel,
        out_type=jax.ShapeDtypeStruct((2 * N, H), jnp.float32),
        mesh=mesh,
        scratch_types=(
            [pltpu.VMEM((C, H), jnp.float32) for _ in range(NRB)]
            + [pltpu.VMEM((C,), jnp.int32) for _ in range(NI)]
            + [pltpu.VMEM((C,), jnp.int32) for _ in range(NI)]
            + [pltpu.VMEM((C,), jnp.float32) for _ in range(NI)]
            + [pltpu.VMEM_SHARED((N, H), jnp.float32)]
            + [pltpu.SemaphoreType.DMA] * (NI + 2 * NRB)
        ),
    )
    def body(ego2_r, src_r, dst_r, w_r, z_r, lap2_r, *sc):
        rows = sc[0:NRB]
        sidx = sc[NRB:NRB + NI]
        didx = sc[NRB + NI:NRB + 2 * NI]
        wvb = sc[NRB + 2 * NI:NRB + 3 * NI]
        acc = sc[NRB + 3 * NI]
        isem = sc[NRB + 3 * NI + 1:NRB + 4 * NI + 1]
        gsem = sc[NRB + 4 * NI + 1:NRB + 4 * NI + 1 + NRB]
        ssem = sc[NRB + 4 * NI + 1 + NRB:NRB + 4 * NI + 1 + 2 * NRB]

        c = lax.axis_index("c")
        s = lax.axis_index("s")

        @pl.when(s < NS - 1)
        def _():
            pltpu.sync_copy(z_r, acc.at[pl.ds(s * BR, BR)])

        @pl.when(s == NS - 1)
        def _():
            pltpu.sync_copy(z_r.at[pl.ds(0, BR_LAST)],
                            acc.at[pl.ds((NS - 1) * BR, BR_LAST)])

        base_e = s * T
        coff = c * N

        def idx_start(g, m):
            off = base_e + g * C
            pltpu.async_copy(src_r.at[pl.ds(off, C)], sidx[m], isem[m])
            pltpu.async_copy(dst_r.at[pl.ds(off, C)], didx[m], isem[m])
            pltpu.async_copy(w_r.at[pl.ds(off, C)], wvb[m], isem[m])

        def idx_wait(g, m):
            off = base_e + g * C
            pltpu.make_async_copy(
                src_r.at[pl.ds(off, C)], sidx[m], isem[m]).wait()
            pltpu.make_async_copy(
                dst_r.at[pl.ds(off, C)], didx[m], isem[m]).wait()
            pltpu.make_async_copy(
                w_r.at[pl.ds(off, C)], wvb[m], isem[m]).wait()

        def add_off(m):
            for k in range(C // L):
                sidx[m][pl.ds(k * L, L)] = sidx[m][pl.ds(k * L, L)] + coff

        def gather_start(b, m):
            pltpu.async_copy(ego2_r.at[sidx[m]], rows[b], gsem[b])

        def gather_wait(b, m):
            pltpu.make_async_copy(
                ego2_r.at[sidx[m]], rows[b], gsem[b]).wait()

        def scat_start(b, m):
            pltpu.async_copy(rows[b], acc.at[didx[m]], ssem[b], add=True)

        def scat_wait(b, m):
            pltpu.make_async_copy(
                rows[b], acc.at[didx[m]], ssem[b]).wait()

        def scale(b, m):
            def group_body(gg, carry):
                w16 = wvb[m][pl.ds(gg * L, L)]
                for ii in range(L):
                    e = gg * L + ii
                    wvec = jnp.full((L,), w16[ii], dtype=jnp.float32)
                    for j in range(H // L):
                        rows[b][e, pl.ds(j * L, L)] = (
                            rows[b][e, pl.ds(j * L, L)] * wvec)
                return carry

            lax.fori_loop(0, C // L, group_body, 0)

        plsc.subcore_barrier()

        idx_start(0, 0)
        idx_start(1, 1)
        idx_start(2, 2)
        idx_wait(0, 0)
        add_off(0)
        gather_start(0, 0)

        nquads = nchunks // NI

        def pipe_body(t, carry):
            for i in range(NI):
                g = NI * t + i
                b = i % NRB
                b2 = (i + 1) % NRB
                m = i
                m_prev = (i + NI - 1) % NI
                m_next = (i + 1) % NI
                m_pref = (i + NI + 3) % NI

                if i == 0:
                    @pl.when(t >= 1)
                    def _():
                        scat_wait(b2, m_prev)
                else:
                    scat_wait(b2, m_prev)

                def _launch():
                    idx_wait(g + 1, m_next)
                    add_off(m_next)
                    gather_start(b2, m_next)

                if i == NI - 1:
                    @pl.when(t < nquads - 1)
                    def _():
                        _launch()
                else:
                    _launch()

                gather_wait(b, m)

                if i == 0:
                    idx_start(g + 3, m_pref)
                else:
                    @pl.when(t < nquads - 1)
                    def _():
                        idx_start(g + 3, m_pref)

                scale(b, m)
                scat_start(b, m)
            return carry

        lax.fori_loop(0, nquads, pipe_body, 0)

        scat_wait((nchunks - 1) % NRB, (nchunks - 1) % NI)

        plsc.subcore_barrier()

        @pl.when(s < NS - 1)
        def _():
            pltpu.sync_copy(acc.at[pl.ds(s * BR, BR)],
                            lap2_r.at[pl.ds(coff + s * BR, BR)])

        @pl.when(s == NS - 1)
        def _():
            pltpu.sync_copy(acc.at[pl.ds((NS - 1) * BR, BR_LAST)],
                            lap2_r.at[pl.ds(coff + (NS - 1) * BR, BR_LAST)])

    return body(ego2, src, dst, w, zrows)


def _tc_dense(ego, lap2, W1, W2, b):
    R = 1000
    nb = N // R

    def body(ego_r, lapl_r, lapr_r, w1_r, w2_r, b_r, out_r):
        el = ego_r[:, :H]
        er = ego_r[:, H:]
        ll = lapl_r[...]
        lr = lapr_r[...]
        acc = jnp.dot(el + ll, w1_r[:H, :], preferred_element_type=jnp.float32)
        acc += jnp.dot(er + lr, w1_r[H:, :], preferred_element_type=jnp.float32)
        acc += jnp.dot(el * ll, w2_r[:H, :], preferred_element_type=jnp.float32)
        acc += jnp.dot(er * lr, w2_r[H:, :], preferred_element_type=jnp.float32)
        pre = acc + b_r[...]
        y = jnp.where(pre >= 0, pre, 0.2 * pre)
        norm = jnp.sqrt(jnp.sum(y * y, axis=1, keepdims=True))
        out_r[...] = y / jnp.maximum(norm, 1e-12)

    return pl.pallas_call(
        body,
        grid=(nb,),
        in_specs=[
            pl.BlockSpec((R, D), lambda i: (i, 0)),
            pl.BlockSpec((R, H), lambda i: (i, 0)),
            pl.BlockSpec((R, H), lambda i: (i + nb, 0)),
            pl.BlockSpec((D, D), lambda i: (0, 0)),
            pl.BlockSpec((D, D), lambda i: (0, 0)),
            pl.BlockSpec((1, D), lambda i: (0, 0)),
        ],
        out_specs=pl.BlockSpec((R, D), lambda i: (i, 0)),
        out_shape=jax.ShapeDtypeStruct((N, D), jnp.float32),
    )(ego, lap2, lap2, W1, W2, b)


def kernel(ego_embeddings, edge_index, edge_weight, W1, W2, b1, b2):
    E = edge_weight.shape[0]

    ego2 = jnp.concatenate(
        [ego_embeddings[:, :H], ego_embeddings[:, H:]], axis=0)

    per_tile = -(-E // (NS * C * NI)) * C * NI
    Ep = per_tile * NS
    pad = Ep - E
    nchunks = per_tile // C
    src = jnp.concatenate([edge_index[0], jnp.zeros((pad,), jnp.int32)])
    dst = jnp.concatenate([edge_index[1], jnp.zeros((pad,), jnp.int32)])
    w = jnp.concatenate([edge_weight, jnp.zeros((pad,), jnp.float32)])

    zrows = jnp.zeros((BR, H), jnp.float32)
    lap2 = _sc_segment_sum(ego2, src, dst, w, zrows, nchunks)

    b = b1 + b2
    return _tc_dense(ego_embeddings, lap2, W1, W2, b)
